# baseline (speedup 1.0000x reference)
import jax
import jax.numpy as jnp
from jax import lax
from jax.experimental import pallas as pl
from jax.experimental.pallas import tpu as pltpu
from jax.experimental.pallas import tpu_sc as plsc

NUM_CORES = 2
NUM_SUBCORES = 16
LANES = 16
NW = NUM_CORES * NUM_SUBCORES

DIM = 32
L = 200
HALF = 100

C = 8


def _bag_kernel(x_hbm, len_hbm, w_hbm, out_hbm,
                idx_v, rows_v, len_v, out_v, gsem0, gsem1):
    B = out_hbm.shape[0]
    bags_per_w = B // NW
    nchunk = bags_per_w // C

    wid = lax.axis_index("s") * NUM_CORES + lax.axis_index("c")
    base = wid * bags_per_w

    pltpu.sync_copy(len_hbm.at[pl.ds(base, bags_per_w)],
                    len_v.at[pl.ds(0, bags_per_w)])

    def fire(slot, k, sem):
        bag0 = base + k * C
        pltpu.sync_copy(x_hbm.at[pl.ds(bag0, C)], idx_v.at[slot])
        for j in range(C):
            for h in range(2):
                pltpu.async_copy(
                    w_hbm.at[idx_v.at[slot, j, h]],
                    rows_v.at[slot, j, pl.ds(h * HALF, HALF)],
                    sem,
                )

    def drain(slot, sem):
        for j in range(C):
            for h in range(2):
                pltpu.make_async_copy(
                    w_hbm.at[idx_v.at[slot, j, h]],
                    rows_v.at[slot, j, pl.ds(h * HALF, HALF)],
                    sem,
                ).wait()

    fire(0, 0, gsem0)

    def chunk_body(k, _):
        cur = k % 2
        bag0 = base + k * C

        @pl.when(k + 1 < nchunk)
        def _():
            @pl.when(cur == 0)
            def _():
                fire(1, k + 1, gsem1)

            @pl.when(cur == 1)
            def _():
                fire(0, k + 1, gsem0)

        @pl.when(cur == 0)
        def _():
            drain(0, gsem0)

        @pl.when(cur == 1)
        def _():
            drain(1, gsem1)

        for j in range(C):

            def row_body(l, accs, j=j):
                a0, a1 = accs
                return (
                    a0 + rows_v[cur, j, l, pl.ds(0, LANES)],
                    a1 + rows_v[cur, j, l, pl.ds(LANES, LANES)],
                )

            zero = jnp.zeros((LANES,), jnp.float32)
            a0, a1 = lax.fori_loop(0, L, row_body, (zero, zero), unroll=8)
            lv = len_v[pl.ds(k * C + j, LANES)][0]
            out_v[j, pl.ds(0, LANES)] = a0 / lv
            out_v[j, pl.ds(LANES, LANES)] = a1 / lv
        pltpu.sync_copy(out_v, out_hbm.at[pl.ds(bag0, C)])
        return ()

    lax.fori_loop(0, nchunk, chunk_body, ())


@jax.jit
def kernel(x, length, emb_weight):
    B = x.shape[0]
    x3 = x.reshape(B, 2, HALF)
    len_f = length.astype(jnp.float32)
    keep = length[0] >= 0
    w_lin = jnp.where(keep, emb_weight, jnp.float32(0))
    x_lin = jnp.where(keep, x3, jnp.int32(0))

    mesh = plsc.VectorSubcoreMesh(core_axis_name="c", subcore_axis_name="s")
    run = pl.kernel(
        _bag_kernel,
        out_type=jax.ShapeDtypeStruct((B, DIM), jnp.float32),
        mesh=mesh,
        scratch_types=[
            pltpu.VMEM((2, C, 2, HALF), jnp.int32),
            pltpu.VMEM((2, C, L, DIM), jnp.float32),
            pltpu.VMEM((B // NW + LANES,), jnp.float32),
            pltpu.VMEM((C, DIM), jnp.float32),
            pltpu.SemaphoreType.DMA,
            pltpu.SemaphoreType.DMA,
        ],
        compiler_params=pltpu.CompilerParams(use_tc_tiling_on_sc=False),
    )
    return run(x_lin, len_f, w_lin)


# speedup vs baseline: 1.5072x; 1.5072x over previous
import jax
import jax.numpy as jnp
from jax import lax
from jax.experimental import pallas as pl
from jax.experimental.pallas import tpu as pltpu
from jax.experimental.pallas import tpu_sc as plsc

NUM_CORES = 2
NUM_SUBCORES = 16
LANES = 16
NW = NUM_CORES * NUM_SUBCORES

DIM = 32
L = 200
HALF = 100

C = 8


def _bag_kernel(x_hbm, len_hbm, w_hbm, out_hbm,
                idx_v, rows_v, len_v, out_v,
                gsem0, gsem1, isem, osem0, osem1):
    B = out_hbm.shape[0]
    bags_per_w = B // NW
    nchunk = bags_per_w // C

    wid = lax.axis_index("s") * NUM_CORES + lax.axis_index("c")
    base = wid * bags_per_w

    pltpu.sync_copy(len_hbm.at[pl.ds(base, bags_per_w)],
                    len_v.at[pl.ds(0, bags_per_w)])

    def start_idx(slot, k):
        pltpu.async_copy(x_hbm.at[pl.ds(base + k * C, C)], idx_v.at[slot], isem)

    def wait_idx(slot, k):
        pltpu.make_async_copy(
            x_hbm.at[pl.ds(base + k * C, C)], idx_v.at[slot], isem
        ).wait()

    def fire(slot, sem):
        for j in range(C):
            for h in range(2):
                pltpu.async_copy(
                    w_hbm.at[idx_v.at[slot, j, h]],
                    rows_v.at[slot, j, pl.ds(h * HALF, HALF)],
                    sem,
                )

    def drain(slot, sem):
        for j in range(C):
            for h in range(2):
                pltpu.make_async_copy(
                    w_hbm.at[idx_v.at[slot, j, h]],
                    rows_v.at[slot, j, pl.ds(h * HALF, HALF)],
                    sem,
                ).wait()

    def out_copy(slot, k):
        pltpu.async_copy(
            out_v.at[slot], out_hbm.at[pl.ds(base + k * C, C)],
            osem0 if slot == 0 else osem1,
        )

    def out_wait(slot, k):
        pltpu.make_async_copy(
            out_v.at[slot], out_hbm.at[pl.ds(base + k * C, C)],
            osem0 if slot == 0 else osem1,
        ).wait()

    start_idx(0, 0)
    wait_idx(0, 0)
    fire(0, gsem0)
    start_idx(1, 1)

    def chunk_body(k, _):
        cur = k % 2
        bag0 = base + k * C

        @pl.when(k + 1 < nchunk)
        def _():
            @pl.when(cur == 0)
            def _():
                wait_idx(1, k + 1)
                fire(1, gsem1)

            @pl.when(cur == 1)
            def _():
                wait_idx(0, k + 1)
                fire(0, gsem0)

        @pl.when(cur == 0)
        def _():
            drain(0, gsem0)

        @pl.when(cur == 1)
        def _():
            drain(1, gsem1)

        @pl.when(k + 2 < nchunk)
        def _():
            @pl.when(cur == 0)
            def _():
                start_idx(0, k + 2)

            @pl.when(cur == 1)
            def _():
                start_idx(1, k + 2)

        @pl.when(k >= 2)
        def _():
            @pl.when(cur == 0)
            def _():
                out_wait(0, k - 2)

            @pl.when(cur == 1)
            def _():
                out_wait(1, k - 2)

        for j in range(C):

            def row_body(i, accs, j=j):
                a0, a1, b0, b1 = accs
                l = i * 2
                a0 = a0 + rows_v[cur, j, l, pl.ds(0, LANES)]
                a1 = a1 + rows_v[cur, j, l, pl.ds(LANES, LANES)]
                b0 = b0 + rows_v[cur, j, l + 1, pl.ds(0, LANES)]
                b1 = b1 + rows_v[cur, j, l + 1, pl.ds(LANES, LANES)]
                return (a0, a1, b0, b1)

            zero = jnp.zeros((LANES,), jnp.float32)
            a0, a1, b0, b1 = lax.fori_loop(
                0, L // 2, row_body, (zero, zero, zero, zero), unroll=4
            )
            lv = len_v[pl.ds(k * C + j, LANES)][0]
            out_v[cur, j, pl.ds(0, LANES)] = (a0 + b0) / lv
            out_v[cur, j, pl.ds(LANES, LANES)] = (a1 + b1) / lv

        @pl.when(cur == 0)
        def _():
            out_copy(0, k)

        @pl.when(cur == 1)
        def _():
            out_copy(1, k)

        return ()

    lax.fori_loop(0, nchunk, chunk_body, ())

    last = nchunk - 1
    out_wait((last - 1) % 2, last - 1)
    out_wait(last % 2, last)


@jax.jit
def kernel(x, length, emb_weight):
    B = x.shape[0]
    x3 = x.reshape(B, 2, HALF)
    len_f = length.astype(jnp.float32)

    mesh = plsc.VectorSubcoreMesh(core_axis_name="c", subcore_axis_name="s")
    run = pl.kernel(
        _bag_kernel,
        out_type=jax.ShapeDtypeStruct((B, DIM), jnp.float32),
        mesh=mesh,
        scratch_types=[
            pltpu.VMEM((2, C, 2, HALF), jnp.int32),
            pltpu.VMEM((2, C, L, DIM), jnp.float32),
            pltpu.VMEM((B // NW + LANES,), jnp.float32),
            pltpu.VMEM((2, C, DIM), jnp.float32),
            pltpu.SemaphoreType.DMA,
            pltpu.SemaphoreType.DMA,
            pltpu.SemaphoreType.DMA,
            pltpu.SemaphoreType.DMA,
            pltpu.SemaphoreType.DMA,
        ],
        compiler_params=pltpu.CompilerParams(use_tc_tiling_on_sc=False),
    )
    return run(x3, len_f, emb_weight)


# speedup vs baseline: 1.5090x; 1.0011x over previous
import jax
import jax.numpy as jnp
from jax import lax
from jax.experimental import pallas as pl
from jax.experimental.pallas import tpu as pltpu
from jax.experimental.pallas import tpu_sc as plsc

NUM_CORES = 2
NUM_SUBCORES = 16
LANES = 16
NW = NUM_CORES * NUM_SUBCORES

DIM = 32
L = 200
HALF = 100

C = 8


def _bag_kernel(x_hbm, len_hbm, w_hbm, out_hbm,
                idx_v, rows_v, len_v, out_v,
                gsem0, gsem1, isem, osem0, osem1):
    B = len_hbm.shape[0]
    bags_per_w = B // NW
    rows_per_chunk = C * DIM // 128
    nchunk = bags_per_w // C

    wid = lax.axis_index("s") * NUM_CORES + lax.axis_index("c")
    base = wid * bags_per_w

    pltpu.sync_copy(len_hbm.at[pl.ds(base, bags_per_w)],
                    len_v.at[pl.ds(0, bags_per_w)])

    def start_idx(slot, k):
        pltpu.async_copy(x_hbm.at[pl.ds(base + k * C, C)], idx_v.at[slot], isem)

    def wait_idx(slot, k):
        pltpu.make_async_copy(
            x_hbm.at[pl.ds(base + k * C, C)], idx_v.at[slot], isem
        ).wait()

    def fire(slot, sem):
        for j in range(C):
            for h in range(2):
                pltpu.async_copy(
                    w_hbm.at[idx_v.at[slot, j, h]],
                    rows_v.at[slot, j, pl.ds(h * HALF, HALF)],
                    sem,
                )

    def drain(slot, sem):
        for j in range(C):
            for h in range(2):
                pltpu.make_async_copy(
                    w_hbm.at[idx_v.at[slot, j, h]],
                    rows_v.at[slot, j, pl.ds(h * HALF, HALF)],
                    sem,
                ).wait()

    def out_copy(slot, k):
        row0 = (base + k * C) * DIM // 128
        pltpu.async_copy(
            out_v.at[slot], out_hbm.at[pl.ds(row0, rows_per_chunk)],
            osem0 if slot == 0 else osem1,
        )

    def out_wait(slot, k):
        row0 = (base + k * C) * DIM // 128
        pltpu.make_async_copy(
            out_v.at[slot], out_hbm.at[pl.ds(row0, rows_per_chunk)],
            osem0 if slot == 0 else osem1,
        ).wait()

    start_idx(0, 0)
    wait_idx(0, 0)
    fire(0, gsem0)
    start_idx(1, 1)

    def chunk_body(k, _):
        cur = k % 2
        bag0 = base + k * C

        @pl.when(k + 1 < nchunk)
        def _():
            @pl.when(cur == 0)
            def _():
                wait_idx(1, k + 1)
                fire(1, gsem1)

            @pl.when(cur == 1)
            def _():
                wait_idx(0, k + 1)
                fire(0, gsem0)

        @pl.when(cur == 0)
        def _():
            drain(0, gsem0)

        @pl.when(cur == 1)
        def _():
            drain(1, gsem1)

        @pl.when(k + 2 < nchunk)
        def _():
            @pl.when(cur == 0)
            def _():
                start_idx(0, k + 2)

            @pl.when(cur == 1)
            def _():
                start_idx(1, k + 2)

        @pl.when(k >= 2)
        def _():
            @pl.when(cur == 0)
            def _():
                out_wait(0, k - 2)

            @pl.when(cur == 1)
            def _():
                out_wait(1, k - 2)

        for j in range(C):

            def row_body(i, accs, j=j):
                a0, a1, b0, b1 = accs
                l = i * 2
                a0 = a0 + rows_v[cur, j, l, pl.ds(0, LANES)]
                a1 = a1 + rows_v[cur, j, l, pl.ds(LANES, LANES)]
                b0 = b0 + rows_v[cur, j, l + 1, pl.ds(0, LANES)]
                b1 = b1 + rows_v[cur, j, l + 1, pl.ds(LANES, LANES)]
                return (a0, a1, b0, b1)

            zero = jnp.zeros((LANES,), jnp.float32)
            a0, a1, b0, b1 = lax.fori_loop(
                0, L // 2, row_body, (zero, zero, zero, zero), unroll=8
            )
            lv = len_v[pl.ds(k * C + j, LANES)][0]
            out_v[cur, j * DIM // 128, pl.ds(j * DIM % 128, LANES)] = (
                a0 + b0
            ) / lv
            out_v[cur, j * DIM // 128, pl.ds(j * DIM % 128 + LANES, LANES)] = (
                a1 + b1
            ) / lv

        @pl.when(cur == 0)
        def _():
            out_copy(0, k)

        @pl.when(cur == 1)
        def _():
            out_copy(1, k)

        return ()

    lax.fori_loop(0, nchunk, chunk_body, ())

    last = nchunk - 1
    out_wait((last - 1) % 2, last - 1)
    out_wait(last % 2, last)


@jax.jit
def kernel(x, length, emb_weight):
    B = x.shape[0]
    x3 = x.reshape(B, 2, HALF)
    len_f = length.astype(jnp.float32)

    mesh = plsc.VectorSubcoreMesh(core_axis_name="c", subcore_axis_name="s")
    run = pl.kernel(
        _bag_kernel,
        out_type=jax.ShapeDtypeStruct((B * DIM // 128, 128), jnp.float32),
        mesh=mesh,
        scratch_types=[
            pltpu.VMEM((2, C, 2, HALF), jnp.int32),
            pltpu.VMEM((2, C, L, DIM), jnp.float32),
            pltpu.VMEM((B // NW + LANES,), jnp.float32),
            pltpu.VMEM((2, C * DIM // 128, 128), jnp.float32),
            pltpu.SemaphoreType.DMA,
            pltpu.SemaphoreType.DMA,
            pltpu.SemaphoreType.DMA,
            pltpu.SemaphoreType.DMA,
            pltpu.SemaphoreType.DMA,
        ],
        compiler_params=pltpu.CompilerParams(use_tc_tiling_on_sc=False),
    )
    return run(x3, len_f, emb_weight).reshape(B, DIM)
